# speedup vs baseline: 1.1841x; 1.1841x over previous
import jax
import jax.numpy as jnp
from jax.experimental import pallas as pl
from jax.experimental.pallas import tpu as pltpu

HIDDEN = 1024
INTER = 704
NUM_EXPERTS = 8
ETOT = 10
T = 2048
TBLK = 1024


def _moe_body(gwT_ref, x32_ref, xb_ref, wg_ref, wu_ref, wd_ref, out_ref, wd16_ref):
    e = pl.program_id(1)

    @pl.when(e == 0)
    def _gate():
        xf = x32_ref[...]
        logits = jnp.dot(xf, gwT_ref[...], preferred_element_type=jnp.float32)
        io8 = jax.lax.broadcasted_iota(jnp.int32, logits.shape, 1)
        m1 = jnp.max(logits, axis=1, keepdims=True)
        a1 = jnp.min(jnp.where(logits == m1, io8, 99), axis=1, keepdims=True)
        lm = jnp.where(io8 == a1, -jnp.inf, logits)
        m2 = jnp.max(lm, axis=1, keepdims=True)
        a2 = jnp.min(jnp.where(lm == m2, io8, 99), axis=1, keepdims=True)
        w1 = 1.0 / (1.0 + jnp.exp(m2 - m1))
        w2 = 1.0 - w1
        io16 = jax.lax.broadcasted_iota(jnp.int32, (TBLK, 16), 1)
        wd16_ref[...] = (
            jnp.where(io16 == a1, w1, 0.0)
            + jnp.where(io16 == a2, w2, 0.0)
            + jnp.where((io16 >= NUM_EXPERTS) & (io16 < ETOT), 1.0, 0.0)
        )

    xb = xb_ref[...]
    g = jnp.dot(xb, wg_ref[0], preferred_element_type=jnp.float32)
    u = jnp.dot(xb, wu_ref[0], preferred_element_type=jnp.float32)
    h = (g / (1.0 + jnp.exp(-g))) * u
    o = jnp.dot(h.astype(jnp.bfloat16), wd_ref[0], preferred_element_type=jnp.float32)
    io16 = jax.lax.broadcasted_iota(jnp.int32, (TBLK, 16), 1)
    wcol = jnp.sum(jnp.where(io16 == e, wd16_ref[...], 0.0), axis=1, keepdims=True)

    @pl.when(e == 0)
    def _init():
        out_ref[...] = wcol * o

    @pl.when(e > 0)
    def _acc():
        out_ref[...] += wcol * o


def kernel(hidden_states, gate_weight, w_gate, w_up, w_down, sw_gate, sw_up, sw_down):
    orig_shape = hidden_states.shape
    x = hidden_states.reshape(T, HIDDEN)
    gwT = gate_weight.T
    xb = x.astype(jnp.bfloat16)
    wg_all = jnp.concatenate(
        [w_gate, sw_gate[:, :INTER][None], sw_gate[:, INTER:][None]], axis=0
    ).astype(jnp.bfloat16)
    wu_all = jnp.concatenate(
        [w_up, sw_up[:, :INTER][None], sw_up[:, INTER:][None]], axis=0
    ).astype(jnp.bfloat16)
    wd_all = jnp.concatenate(
        [w_down, sw_down[None, :INTER, :], sw_down[None, INTER:, :]], axis=0
    ).astype(jnp.bfloat16)

    out = pl.pallas_call(
        _moe_body,
        grid=(T // TBLK, ETOT),
        in_specs=[
            pl.BlockSpec((HIDDEN, NUM_EXPERTS), lambda t, e: (0, 0)),
            pl.BlockSpec((TBLK, HIDDEN), lambda t, e: (t, 0)),
            pl.BlockSpec((TBLK, HIDDEN), lambda t, e: (t, 0)),
            pl.BlockSpec((1, HIDDEN, INTER), lambda t, e: (e, 0, 0)),
            pl.BlockSpec((1, HIDDEN, INTER), lambda t, e: (e, 0, 0)),
            pl.BlockSpec((1, INTER, HIDDEN), lambda t, e: (e, 0, 0)),
        ],
        out_specs=pl.BlockSpec((TBLK, HIDDEN), lambda t, e: (t, 0)),
        out_shape=jax.ShapeDtypeStruct((T, HIDDEN), jnp.float32),
        scratch_shapes=[pltpu.VMEM((TBLK, 16), jnp.float32)],
    )(gwT, x, xb, wg_all, wu_all, wd_all)
    return out.reshape(orig_shape)


# speedup vs baseline: 1.2126x; 1.0241x over previous
import functools

import jax
import jax.numpy as jnp
from jax import lax
from jax.experimental import pallas as pl
from jax.experimental.pallas import tpu as pltpu
from jax.experimental.pallas import tpu_sc as plsc

T = 2048
D = 1024
F = 704
E = 8
BLK = 256
NRB = 24
RS = NRB * BLK
XS = RS + T
NB = NRB + 2 * (T // BLK)
OS = NB * BLK
NW = 32
TPW = T // NW


@functools.cache
def _get_mesh():
    return plsc.VectorSubcoreMesh(core_axis_name="c", subcore_axis_name="s")


def _route_body(x_ref, gwT_ref, slot0_ref, slot1_ref, wts_ref, be_ref):
    xf = x_ref[...]
    logits = jnp.dot(xf, gwT_ref[...], preferred_element_type=jnp.float32)
    io8 = lax.broadcasted_iota(jnp.int32, logits.shape, 1)
    m1 = jnp.max(logits, axis=1, keepdims=True)
    a1 = jnp.min(jnp.where(logits == m1, io8, 99), axis=1, keepdims=True)
    lm = jnp.where(io8 == a1, -jnp.inf, logits)
    m2 = jnp.max(lm, axis=1, keepdims=True)
    a2 = jnp.min(jnp.where(lm == m2, io8, 99), axis=1, keepdims=True)
    w1 = 1.0 / (1.0 + jnp.exp(m2 - m1))
    wts_ref[...] = jnp.concatenate([w1, 1.0 - w1], axis=1)

    io16 = lax.broadcasted_iota(jnp.int32, (T, 16), 1)
    oh0 = (io16 == a1).astype(jnp.float32)
    oh1 = (io16 == a2).astype(jnp.float32)

    r_io = lax.broadcasted_iota(jnp.int32, (T, T), 0)
    c_io = lax.broadcasted_iota(jnp.int32, (T, T), 1)
    ltri = (r_io > c_io).astype(jnp.bfloat16)
    pre0 = jnp.dot(ltri, oh0.astype(jnp.bfloat16),
                   preferred_element_type=jnp.float32)
    cs0 = jnp.sum(oh0, axis=0, keepdims=True)
    pre1 = jnp.dot(ltri, oh1.astype(jnp.bfloat16),
                   preferred_element_type=jnp.float32) + cs0
    counts = (cs0 + jnp.sum(oh1, axis=0, keepdims=True)).astype(jnp.int32)

    pc = (counts + (BLK - 1)) & ~(BLK - 1)
    u_r = lax.broadcasted_iota(jnp.int32, (16, 16), 0)
    u_c = lax.broadcasted_iota(jnp.int32, (16, 16), 1)
    utri = (u_r < u_c).astype(jnp.float32)
    P = jnp.dot(pc.astype(jnp.float32), utri,
                preferred_element_type=jnp.float32)

    r0 = jnp.sum(jnp.where(io16 == a1, pre0, 0.0), axis=1, keepdims=True)
    r1 = jnp.sum(jnp.where(io16 == a2, pre1, 0.0), axis=1, keepdims=True)
    base0 = jnp.sum(jnp.where(io16 == a1, P, 0.0), axis=1, keepdims=True)
    base1 = jnp.sum(jnp.where(io16 == a2, P, 0.0), axis=1, keepdims=True)
    slot0_ref[...] = (base0 + r0).astype(jnp.int32)
    slot1_ref[...] = (base1 + r1).astype(jnp.int32)

    bid = lax.broadcasted_iota(jnp.int32, (1, 64), 1)
    acc = jnp.zeros((1, 64), jnp.int32)
    for e in range(E):
        pe = P[:, e:e + 1]
        acc = acc + (bid * BLK >= pe.astype(jnp.int32)).astype(jnp.int32)
    be = jnp.clip(acc - 1, 0, E - 1)
    be = jnp.where(bid >= NRB, E, be)
    be = jnp.where(bid >= NRB + T // BLK, E + 1, be)
    be = jnp.where(bid >= NB, 0, be)
    be_ref[...] = be


def _route(x, gwT):
    return pl.pallas_call(
        _route_body,
        grid=(1,),
        in_specs=[
            pl.BlockSpec((T, D), lambda i: (0, 0)),
            pl.BlockSpec((D, E), lambda i: (0, 0)),
        ],
        out_specs=[
            pl.BlockSpec((T, 1), lambda i: (0, 0)),
            pl.BlockSpec((T, 1), lambda i: (0, 0)),
            pl.BlockSpec((T, 2), lambda i: (0, 0)),
            pl.BlockSpec((1, 64), lambda i: (0, 0)),
        ],
        out_shape=[
            jax.ShapeDtypeStruct((T, 1), jnp.int32),
            jax.ShapeDtypeStruct((T, 1), jnp.int32),
            jax.ShapeDtypeStruct((T, 2), jnp.float32),
            jax.ShapeDtypeStruct((1, 64), jnp.int32),
        ],
    )(x, gwT)


def _dispatch_body(x_hbm, s0_hbm, s1_hbm, xs_hbm, idx_v, rows_v, sem):
    wid = lax.axis_index("s") * 2 + lax.axis_index("c")
    base = wid * TPW
    pltpu.sync_copy(x_hbm.at[pl.ds(base, TPW)], rows_v)
    pltpu.sync_copy(s0_hbm.at[pl.ds(base, TPW)], idx_v)
    pltpu.async_copy(rows_v, xs_hbm.at[idx_v], sem).wait()
    pltpu.sync_copy(s1_hbm.at[pl.ds(base, TPW)], idx_v)
    pltpu.async_copy(rows_v, xs_hbm.at[idx_v], sem).wait()
    pltpu.sync_copy(rows_v, xs_hbm.at[pl.ds(RS + base, TPW)])


def _dispatch(x, s0, s1):
    f = pl.kernel(
        _dispatch_body,
        mesh=_get_mesh(),
        out_type=[jax.ShapeDtypeStruct((XS, D), jnp.float32)],
        scratch_types=[
            pltpu.VMEM((TPW,), jnp.int32),
            pltpu.VMEM((TPW, D), jnp.float32),
            pltpu.SemaphoreType.DMA,
        ],
    )
    return f(x, s0, s1)


def _gemm_body(be_ref, xs_ref, wg_ref, wu_ref, wd_ref, os_ref):
    xb = xs_ref[...].astype(jnp.bfloat16)
    g = jnp.dot(xb, wg_ref[0], preferred_element_type=jnp.float32)
    u = jnp.dot(xb, wu_ref[0], preferred_element_type=jnp.float32)
    h = (g / (1.0 + jnp.exp(-g))) * u
    os_ref[...] = jnp.dot(h.astype(jnp.bfloat16), wd_ref[0],
                          preferred_element_type=jnp.float32)


def _gemm(xs, wg_all, wu_all, wd_all, be):
    grid_spec = pltpu.PrefetchScalarGridSpec(
        num_scalar_prefetch=1,
        grid=(NB,),
        in_specs=[
            pl.BlockSpec((BLK, D),
                         lambda i, be: (jnp.where(i >= NRB + T // BLK,
                                                  i - T // BLK, i), 0)),
            pl.BlockSpec((1, D, F), lambda i, be: (be[i], 0, 0)),
            pl.BlockSpec((1, D, F), lambda i, be: (be[i], 0, 0)),
            pl.BlockSpec((1, F, D), lambda i, be: (be[i], 0, 0)),
        ],
        out_specs=pl.BlockSpec((BLK, D), lambda i, be: (i, 0)),
    )
    return pl.pallas_call(
        _gemm_body,
        grid_spec=grid_spec,
        out_shape=jax.ShapeDtypeStruct((OS, D), jnp.float32),
    )(be, xs, wg_all, wu_all, wd_all)


def _combine_body(os_hbm, s0_hbm, s1_hbm, z_hbm, idx_v, rows_v, sem):
    wid = lax.axis_index("s") * 2 + lax.axis_index("c")
    base = wid * TPW
    pltpu.sync_copy(s0_hbm.at[pl.ds(base, TPW)], idx_v)
    pltpu.async_copy(os_hbm.at[idx_v], rows_v, sem).wait()
    pltpu.sync_copy(rows_v, z_hbm.at[pl.ds(base, TPW)])
    pltpu.sync_copy(s1_hbm.at[pl.ds(base, TPW)], idx_v)
    pltpu.async_copy(os_hbm.at[idx_v], rows_v, sem).wait()
    pltpu.sync_copy(rows_v, z_hbm.at[pl.ds(T + base, TPW)])


def _combine(os_, s0, s1):
    f = pl.kernel(
        _combine_body,
        mesh=_get_mesh(),
        out_type=[jax.ShapeDtypeStruct((2 * T, D), jnp.float32)],
        scratch_types=[
            pltpu.VMEM((TPW,), jnp.int32),
            pltpu.VMEM((TPW, D), jnp.float32),
            pltpu.SemaphoreType.DMA,
        ],
    )
    return f(os_, s0, s1)


FBLK = 256


def _final_body(z0_ref, z1_ref, sh0_ref, sh1_ref, wts_ref, y_ref):
    w1 = wts_ref[:, 0:1]
    w2 = wts_ref[:, 1:2]
    y_ref[...] = (w1 * z0_ref[...] + w2 * z1_ref[...]
                  + sh0_ref[...] + sh1_ref[...])


def _final(z, os_, wts):
    nblk = T // FBLK
    sh_base = RS // FBLK
    return pl.pallas_call(
        _final_body,
        grid=(nblk,),
        in_specs=[
            pl.BlockSpec((FBLK, D), lambda i: (i, 0)),
            pl.BlockSpec((FBLK, D), lambda i: (i + T // FBLK, 0)),
            pl.BlockSpec((FBLK, D), lambda i: (i + sh_base, 0)),
            pl.BlockSpec((FBLK, D), lambda i: (i + sh_base + T // FBLK, 0)),
            pl.BlockSpec((FBLK, 2), lambda i: (i, 0)),
        ],
        out_specs=pl.BlockSpec((FBLK, D), lambda i: (i, 0)),
        out_shape=jax.ShapeDtypeStruct((T, D), jnp.float32),
    )(z, z, os_, os_, wts)


def kernel(hidden_states, gate_weight, w_gate, w_up, w_down, sw_gate, sw_up, sw_down):
    orig_shape = hidden_states.shape
    x = hidden_states.reshape(T, D)
    gwT = gate_weight.T

    slot0, slot1, wts, be64 = _route(x, gwT)
    s0 = slot0.reshape(T)
    s1 = slot1.reshape(T)
    be = be64.reshape(64)[:NB]

    (xs,) = _dispatch(x, s0, s1)

    wg_all = jnp.concatenate(
        [w_gate, sw_gate[:, :F][None], sw_gate[:, F:][None]], axis=0
    ).astype(jnp.bfloat16)
    wu_all = jnp.concatenate(
        [w_up, sw_up[:, :F][None], sw_up[:, F:][None]], axis=0
    ).astype(jnp.bfloat16)
    wd_all = jnp.concatenate(
        [w_down, sw_down[None, :F, :], sw_down[None, F:, :]], axis=0
    ).astype(jnp.bfloat16)

    os_ = _gemm(xs, wg_all, wu_all, wd_all, be)

    (z,) = _combine(os_, s0, s1)

    y = _final(z, os_, wts)
    return y.reshape(orig_shape)
